# Initial kernel scaffold; baseline (speedup 1.0000x reference)
import jax
import jax.numpy as jnp
from jax.experimental import pallas as pl


def kernel(x, token_table, pos_table):
    raise NotImplementedError("write your pallas kernel here")


# speedup vs baseline: 3.8809x; 3.8809x over previous
import functools

import jax
import jax.numpy as jnp
from jax import lax
from jax.experimental import pallas as pl
from jax.experimental.pallas import tpu as pltpu
from jax.experimental.pallas import tpu_sc as plsc

D_MODEL = 64
SEQ = 200
NC, NS = 2, 16
NW = NC * NS
CHR = 100
LANES = 16
VPR = D_MODEL // LANES


def _emb_body(x_hbm, table_hbm, pos_hbm, out_hbm,
              idx_v, pos_v, gbuf, obuf,
              gsem0, gsem1, osem0, osem1):
    nch = x_hbm.shape[1]
    wid = lax.axis_index("s") * NC + lax.axis_index("c")

    pltpu.sync_copy(x_hbm.at[wid], idx_v)
    pltpu.sync_copy(pos_hbm, pos_v)

    gsems = (gsem0, gsem1)
    osems = (osem0, osem1)

    def gather_start(c, buf):
        return pltpu.async_copy(
            table_hbm.at[idx_v.at[c]], gbuf.at[buf], gsems[buf])

    def out_start(c, buf):
        return pltpu.async_copy(
            obuf.at[buf], out_hbm.at[wid * nch + c], osems[buf])

    gather_start(0, 0)
    gather_start(1, 1)

    def chunk(t, b):
        c = 2 * t + b
        pltpu.make_async_copy(
            table_hbm.at[idx_v.at[c]], gbuf.at[b], gsems[b]).wait()
        @pl.when(t > 0)
        def _():
            pltpu.make_async_copy(
                obuf.at[b], out_hbm.at[wid * nch + (c - 2)], osems[b]).wait()

        s_base = b * CHR

        def add_rows(r, _):
            for u in range(2):
                rr = 2 * r + u
                for j in range(VPR):
                    sl = pl.ds(j * LANES, LANES)
                    obuf[b, rr, sl] = gbuf[b, rr, sl] + pos_v[s_base + rr, sl]
            return 0

        lax.fori_loop(0, CHR // 2, add_rows, 0)

        @pl.when(c + 2 < nch)
        def _():
            gather_start(c + 2, b)

        out_start(c, b)

    def step(t, _):
        chunk(t, 0)
        chunk(t, 1)
        return 0

    lax.fori_loop(0, nch // 2, step, 0)

    for b in range(2):
        c_last = nch - 2 + b
        pltpu.make_async_copy(
            obuf.at[b], out_hbm.at[wid * nch + c_last], osems[b]).wait()


def kernel(x, token_table, pos_table):
    B, S = x.shape
    total = B * S
    rows_w = total // NW
    nch = rows_w // CHR
    x3 = x.astype(jnp.int32).reshape(NW, nch, CHR)

    mesh = plsc.VectorSubcoreMesh(core_axis_name="c", subcore_axis_name="s")
    out = pl.kernel(
        _emb_body,
        out_type=jax.ShapeDtypeStruct((NW * nch, CHR, D_MODEL), jnp.float32),
        mesh=mesh,
        compiler_params=pltpu.CompilerParams(use_tc_tiling_on_sc=False),
        scratch_types=[
            pltpu.VMEM((nch, CHR), jnp.int32),
            pltpu.VMEM((SEQ, D_MODEL), jnp.float32),
            pltpu.VMEM((2, CHR, D_MODEL), jnp.float32),
            pltpu.VMEM((2, CHR, D_MODEL), jnp.float32),
            pltpu.SemaphoreType.DMA,
            pltpu.SemaphoreType.DMA,
            pltpu.SemaphoreType.DMA,
            pltpu.SemaphoreType.DMA,
        ],
    )(x3, token_table, pos_table)
    return out.reshape(B, S, D_MODEL)
